# baseline (speedup 1.0000x reference)
import functools

import jax
import jax.numpy as jnp
from jax import lax
from jax.experimental import pallas as pl
from jax.experimental.pallas import tpu as pltpu
from jax.experimental.pallas import tpu_sc as plsc

_NUM_CORES = 2
_NUM_SUBCORES = 16
_NW = _NUM_CORES * _NUM_SUBCORES

_CHUNK = 128
_LANES = 16


@functools.partial(jax.jit, static_argnames=("hist", "batch", "dim"))
def _sc_gather(table, xt, *, hist, batch, dim):
    chunks = hist

    mesh = plsc.VectorSubcoreMesh(
        core_axis_name="c", subcore_axis_name="s", num_cores=_NUM_CORES
    )

    @functools.partial(
        pl.kernel,
        mesh=mesh,
        compiler_params=pltpu.CompilerParams(
            use_tc_tiling_on_sc=False, needs_layout_passes=False
        ),
        out_type=jax.ShapeDtypeStruct(
            (hist, dim // 8, batch // _CHUNK, 8, _CHUNK), table.dtype
        ),
        scratch_types=(
            [pltpu.VMEM((hist, _CHUNK), jnp.int32)]
            + [pltpu.VMEM((_CHUNK, dim), table.dtype) for _ in range(2)]
            + [pltpu.VMEM((dim // 8, 8, _CHUNK), table.dtype) for _ in range(2)]
            + [pltpu.SemaphoreType.DMA for _ in range(4)]
        ),
    )
    def run(table_hbm, xt_hbm, out_hbm, x_v, *rest):
        bufs = rest[0:2]
        stagings = rest[2:4]
        gsems = rest[4:6]
        osems = rest[6:8]

        wid = lax.axis_index("s") * _NUM_CORES + lax.axis_index("c")
        col = pl.multiple_of(wid * _CHUNK, 8)
        pltpu.sync_copy(xt_hbm.at[:, pl.ds(col, _CHUNK)], x_v)

        def gather(o, b):
            return pltpu.make_async_copy(
                table_hbm.at[x_v.at[o]], bufs[b], gsems[b]
            )

        def transpose(b):
            kvs = [
                lax.iota(jnp.int32, _LANES) + g * _LANES
                for g in range(_CHUNK // _LANES)
            ]

            def dbody(d, _):
                zv = jnp.full((_LANES,), 0, jnp.int32)
                dv = zv + d
                dhi = zv + (d >> 3)
                dlo = zv + lax.bitwise_and(d, 7)
                for g in range(_CHUNK // _LANES):
                    vals = plsc.load_gather(bufs[b], [kvs[g], dv])
                    plsc.store_scatter(stagings[b], [dhi, dlo, kvs[g]], vals)
                return 0

            lax.fori_loop(0, dim, dbody, 0)

        def out_copy(o, b):
            return pltpu.make_async_copy(
                stagings[b],
                out_hbm.at[o, :, wid],
                osems[b],
            )

        gather(0, 0).start()

        def step(o, b):
            gather(o, b).wait()

            @pl.when(o + 1 < chunks)
            def _():
                gather(o + 1, 1 - b).start()

            @pl.when(o >= 2)
            def _():
                out_copy(o - 2, b).wait()

            transpose(b)
            out_copy(o, b).start()

        def pair(oo, carry):
            o = oo * 2
            step(o, 0)
            step(o + 1, 1)
            return carry

        lax.fori_loop(0, chunks // 2, pair, 0)
        out_copy(chunks - 2, 0).wait()
        out_copy(chunks - 1, 1).wait()

    return run(table, xt)


def kernel(x, lengths, table):
    del lengths
    batch, hist = x.shape
    vocab, dim = table.shape
    xt = x.T.astype(jnp.int32)
    out = _sc_gather(table, xt, hist=hist, batch=batch, dim=dim)
    return out.transpose(2, 4, 0, 1, 3).reshape(batch, hist, dim)


# speedup vs baseline: 1.7265x; 1.7265x over previous
import functools

import jax
import jax.numpy as jnp
from jax import lax
from jax.experimental import pallas as pl
from jax.experimental.pallas import tpu as pltpu
from jax.experimental.pallas import tpu_sc as plsc

_NUM_CORES = 2
_NUM_SUBCORES = 16
_NW = _NUM_CORES * _NUM_SUBCORES

_CHUNK = 128
_LANES = 16


@functools.partial(jax.jit, static_argnames=("hist", "batch", "dim"))
def _sc_gather(table, xt, *, hist, batch, dim):
    chunks = hist

    mesh = plsc.VectorSubcoreMesh(
        core_axis_name="c", subcore_axis_name="s", num_cores=_NUM_CORES
    )

    @functools.partial(
        pl.kernel,
        mesh=mesh,
        compiler_params=pltpu.CompilerParams(
            use_tc_tiling_on_sc=False, needs_layout_passes=False
        ),
        out_type=jax.ShapeDtypeStruct(
            (hist, dim // 8, batch // _CHUNK, 8, _CHUNK), table.dtype
        ),
        scratch_types=(
            [pltpu.VMEM((hist, _CHUNK), jnp.int32)]
            + [pltpu.VMEM((_CHUNK, dim), table.dtype) for _ in range(2)]
            + [pltpu.VMEM((dim // 8, 8, _CHUNK + 1), table.dtype) for _ in range(2)]
            + [pltpu.SemaphoreType.DMA for _ in range(4)]
        ),
    )
    def run(table_hbm, xt_hbm, out_hbm, x_v, *rest):
        bufs = rest[0:2]
        stagings = rest[2:4]
        gsems = rest[4:6]
        osems = rest[6:8]

        wid = lax.axis_index("s") * _NUM_CORES + lax.axis_index("c")
        col = pl.multiple_of(wid * _CHUNK, 8)
        pltpu.sync_copy(xt_hbm.at[:, pl.ds(col, _CHUNK)], x_v)

        def gather(o, b):
            return pltpu.make_async_copy(
                table_hbm.at[x_v.at[o]], bufs[b], gsems[b]
            )

        def transpose(b):
            dvs = [
                lax.iota(jnp.int32, _LANES) + g * _LANES
                for g in range(dim // _LANES)
            ]
            dhis = [dv >> 3 for dv in dvs]
            dlos = [lax.bitwise_and(dv, 7) for dv in dvs]

            def kbody(k, _):
                kv = jnp.full((_LANES,), 0, jnp.int32) + k
                for g in range(dim // _LANES):
                    vals = plsc.load_gather(bufs[b], [kv, dvs[g]])
                    plsc.store_scatter(stagings[b], [dhis[g], dlos[g], kv], vals)
                return 0

            lax.fori_loop(0, _CHUNK, kbody, 0)

        def out_copy(o, b):
            return pltpu.make_async_copy(
                stagings[b].at[:, :, pl.ds(0, _CHUNK)],
                out_hbm.at[o, :, wid],
                osems[b],
            )

        gather(0, 0).start()

        def step(o, b):
            gather(o, b).wait()

            @pl.when(o + 1 < chunks)
            def _():
                gather(o + 1, 1 - b).start()

            @pl.when(o >= 2)
            def _():
                out_copy(o - 2, b).wait()

            transpose(b)
            out_copy(o, b).start()

        def pair(oo, carry):
            o = oo * 2
            step(o, 0)
            step(o + 1, 1)
            return carry

        lax.fori_loop(0, chunks // 2, pair, 0)
        out_copy(chunks - 2, 0).wait()
        out_copy(chunks - 1, 1).wait()

    return run(table, xt)


def kernel(x, lengths, table):
    del lengths
    batch, hist = x.shape
    vocab, dim = table.shape
    xt = x.T.astype(jnp.int32)
    out = _sc_gather(table, xt, hist=hist, batch=batch, dim=dim)
    return out.transpose(2, 4, 0, 1, 3).reshape(batch, hist, dim)


# speedup vs baseline: 1.7601x; 1.0194x over previous
import functools

import jax
import jax.numpy as jnp
from jax import lax
from jax.experimental import pallas as pl
from jax.experimental.pallas import tpu as pltpu
from jax.experimental.pallas import tpu_sc as plsc

_NUM_CORES = 2
_NUM_SUBCORES = 16
_NW = _NUM_CORES * _NUM_SUBCORES

_CHUNK = 128
_LANES = 16


@functools.partial(jax.jit, static_argnames=("hist", "batch", "dim"))
def _sc_gather(table, xt, *, hist, batch, dim):
    chunks = hist

    mesh = plsc.VectorSubcoreMesh(
        core_axis_name="c", subcore_axis_name="s", num_cores=_NUM_CORES
    )

    @functools.partial(
        pl.kernel,
        mesh=mesh,
        compiler_params=pltpu.CompilerParams(
            use_tc_tiling_on_sc=False, needs_layout_passes=False
        ),
        out_type=jax.ShapeDtypeStruct(
            (hist, dim // 8, batch // _CHUNK, 8, _CHUNK), table.dtype
        ),
        scratch_types=(
            [pltpu.VMEM((hist, _CHUNK), jnp.int32)]
            + [pltpu.VMEM((_CHUNK, dim), table.dtype) for _ in range(2)]
            + [pltpu.VMEM((dim // 8, 8, _CHUNK + 1), table.dtype) for _ in range(2)]
            + [pltpu.SemaphoreType.DMA for _ in range(4)]
        ),
    )
    def run(table_hbm, xt_hbm, out_hbm, x_v, *rest):
        bufs = rest[0:2]
        stagings = rest[2:4]
        gsems = rest[4:6]
        osems = rest[6:8]

        wid = lax.axis_index("s") * _NUM_CORES + lax.axis_index("c")
        col = pl.multiple_of(wid * _CHUNK, 8)
        pltpu.sync_copy(xt_hbm.at[:, pl.ds(col, _CHUNK)], x_v)

        def gather(o, b):
            return pltpu.make_async_copy(
                table_hbm.at[x_v.at[o]], bufs[b], gsems[b]
            )

        def transpose(b):
            dvs = [
                lax.iota(jnp.int32, _LANES) + g * _LANES
                for g in range(dim // _LANES)
            ]
            dhis = [dv >> 3 for dv in dvs]
            dlos = [lax.bitwise_and(dv, 7) for dv in dvs]

            def kbody(k, _):
                kv = jnp.full((_LANES,), 0, jnp.int32) + k
                for g in range(dim // _LANES):
                    vals = plsc.load_gather(bufs[b], [kv, dvs[g]])
                    plsc.store_scatter(stagings[b], [dhis[g], dlos[g], kv], vals)
                return 0

            lax.fori_loop(0, _CHUNK, kbody, 0, unroll=8)

        def out_copy(o, b):
            return pltpu.make_async_copy(
                stagings[b].at[:, :, pl.ds(0, _CHUNK)],
                out_hbm.at[o, :, wid],
                osems[b],
            )

        gather(0, 0).start()

        def step(o, b):
            gather(o, b).wait()

            @pl.when(o + 1 < chunks)
            def _():
                gather(o + 1, 1 - b).start()

            @pl.when(o >= 2)
            def _():
                out_copy(o - 2, b).wait()

            transpose(b)
            out_copy(o, b).start()

        def pair(oo, carry):
            o = oo * 2
            step(o, 0)
            step(o + 1, 1)
            return carry

        lax.fori_loop(0, chunks // 2, pair, 0)
        out_copy(chunks - 2, 0).wait()
        out_copy(chunks - 1, 1).wait()

    return run(table, xt)


def kernel(x, lengths, table):
    del lengths
    batch, hist = x.shape
    vocab, dim = table.shape
    xt = x.T.astype(jnp.int32)
    out = _sc_gather(table, xt, hist=hist, batch=batch, dim=dim)
    return out.transpose(2, 4, 0, 1, 3).reshape(batch, hist, dim)
